# Initial kernel scaffold; baseline (speedup 1.0000x reference)
import jax
import jax.numpy as jnp
from jax import lax
from jax.experimental import pallas as pl
from jax.experimental.pallas import tpu as pltpu
from jax.experimental.pallas import tpu_sc as plsc

N = 10000
E = 320000
D = 128
N_PAD = 10240
NC = 2
NS = 16
NW = NC * NS
EPW = E // NW
CHUNK = 80
NCH = EPW // CHUNK
ROWS_PER_TILE = N_PAD // NS

_mesh = plsc.VectorSubcoreMesh(core_axis_name="c", subcore_axis_name="s")


def _zero_shared(zbuf, shared, sub, rows, width):
    zr = zbuf.shape[0]
    for r in range(zr):
        for dblk in range(width // 16):
            zbuf[r, pl.ds(dblk * 16, 16)] = jnp.zeros((16,), jnp.float32)
    for t in range(rows // zr):
        pltpu.sync_copy(zbuf, shared.at[pl.ds(sub * rows + t * zr, zr)])


def _sc_deg(col2d, ew2d):
    def body(col_hbm, ew_hbm, out_hbm, acc_sh, col_v, ew_v, pay_v, zbuf):
        c = lax.axis_index("c")
        s = lax.axis_index("s")
        wid = c * NS + s
        pltpu.sync_copy(col_hbm.at[wid], col_v)
        pltpu.sync_copy(ew_hbm.at[wid], ew_v)
        _zero_shared(zbuf, acc_sh, s, ROWS_PER_TILE, 16)
        plsc.subcore_barrier()

        def chunk(j, carry):
            for e in range(CHUNK):
                sv = plsc.load_gather(
                    ew_v,
                    [jnp.full((16,), j, jnp.int32), jnp.full((16,), e, jnp.int32)],
                )
                pay_v[e, :] = sv
            pltpu.sync_copy(pay_v, acc_sh.at[col_v.at[j]], add=True)
            return carry

        lax.fori_loop(0, NCH, chunk, 0)
        plsc.subcore_barrier()
        pltpu.sync_copy(
            acc_sh.at[pl.ds(s * ROWS_PER_TILE, ROWS_PER_TILE)],
            out_hbm.at[c, pl.ds(s * ROWS_PER_TILE, ROWS_PER_TILE)],
        )

    k = pl.kernel(
        body,
        out_type=jax.ShapeDtypeStruct((NC, N_PAD, 16), jnp.float32),
        mesh=_mesh,
        scratch_types=[
            pltpu.VMEM_SHARED((N_PAD, 16), jnp.float32),
            pltpu.VMEM((NCH, CHUNK), jnp.int32),
            pltpu.VMEM((NCH, CHUNK), jnp.float32),
            pltpu.VMEM((CHUNK, 16), jnp.float32),
            pltpu.VMEM((16, 16), jnp.float32),
        ],
    )
    return k(col2d, ew2d)


def _sc_agg(row2d, col2d, ew2d, Xs):
    def body(row_hbm, col_hbm, ew_hbm, xs_hbm, out_hbm,
             acc_sh, row_v, col_v, ew_v, rows_v, zbuf, gsem):
        c = lax.axis_index("c")
        s = lax.axis_index("s")
        wid = c * NS + s
        pltpu.sync_copy(row_hbm.at[wid], row_v)
        pltpu.sync_copy(col_hbm.at[wid], col_v)
        pltpu.sync_copy(ew_hbm.at[wid], ew_v)
        _zero_shared(zbuf, acc_sh, s, ROWS_PER_TILE, D)
        plsc.subcore_barrier()

        def chunk(j, carry):
            pltpu.async_copy(xs_hbm.at[row_v.at[j]], rows_v, gsem).wait()
            for e in range(CHUNK):
                sv = plsc.load_gather(
                    ew_v,
                    [jnp.full((16,), j, jnp.int32), jnp.full((16,), e, jnp.int32)],
                )
                for dblk in range(D // 16):
                    sl = pl.ds(dblk * 16, 16)
                    rows_v[e, sl] = rows_v[e, sl] * sv
            pltpu.sync_copy(rows_v, acc_sh.at[col_v.at[j]], add=True)
            return carry

        lax.fori_loop(0, NCH, chunk, 0)
        plsc.subcore_barrier()
        pltpu.sync_copy(
            acc_sh.at[pl.ds(s * ROWS_PER_TILE, ROWS_PER_TILE)],
            out_hbm.at[c, pl.ds(s * ROWS_PER_TILE, ROWS_PER_TILE)],
        )

    k = pl.kernel(
        body,
        out_type=jax.ShapeDtypeStruct((NC, N_PAD, D), jnp.float32),
        mesh=_mesh,
        scratch_types=[
            pltpu.VMEM_SHARED((N_PAD, D), jnp.float32),
            pltpu.VMEM((NCH, CHUNK), jnp.int32),
            pltpu.VMEM((NCH, CHUNK), jnp.int32),
            pltpu.VMEM((NCH, CHUNK), jnp.float32),
            pltpu.VMEM((CHUNK, D), jnp.float32),
            pltpu.VMEM((16, D), jnp.float32),
            pltpu.SemaphoreType.DMA,
        ],
    )
    return k(row2d, col2d, ew2d, Xs)


_BLK = 1000
_GRID = N // _BLK


def _tc_prescale_body(deg_ref, x_ref, xs_ref):
    deg = 1.0 + deg_ref[0, :, :1] + deg_ref[1, :, :1]
    dinv = lax.rsqrt(deg)
    xs_ref[...] = x_ref[...] * dinv


def _tc_prescale(deg_parts, X):
    return pl.pallas_call(
        _tc_prescale_body,
        grid=(_GRID,),
        in_specs=[
            pl.BlockSpec((NC, _BLK, 16), lambda i: (0, i, 0)),
            pl.BlockSpec((_BLK, D), lambda i: (i, 0)),
        ],
        out_specs=pl.BlockSpec((_BLK, D), lambda i: (i, 0)),
        out_shape=jax.ShapeDtypeStruct((N, D), jnp.float32),
    )(deg_parts, X)


def _dot(a, b):
    return lax.dot_general(
        a, b, (((1,), (0,)), ((), ())),
        precision=lax.Precision.HIGHEST,
        preferred_element_type=jnp.float32,
    )


def _tc_dense_body(agg_ref, deg_ref, xs_ref, wz_ref, bz_ref, wh_ref, bh_ref,
                   lwz_ref, lbz_ref, lwh_ref, lbh_ref, out_ref):
    deg = 1.0 + deg_ref[0, :, :1] + deg_ref[1, :, :1]
    dinv = lax.rsqrt(deg)
    p = dinv * (agg_ref[0] + agg_ref[1] + xs_ref[...])
    az = lwz_ref[:D, :]
    ah = lwh_ref[:D, :]
    mz = _dot(wz_ref[...], az)
    mh = _dot(wh_ref[...], ah)
    cz = _dot(bz_ref[...], az) + lbz_ref[...]
    ch = _dot(bh_ref[...], ah) + lbh_ref[...]
    z = jax.nn.sigmoid(_dot(p, mz) + cz)
    ht = jnp.tanh(_dot(p, mh) + ch)
    out_ref[...] = (1.0 - z) * ht


def _tc_dense(agg_parts, deg_parts, Xs, W_z, b_z, W_h, b_h, LW_z, Lb_z, LW_h, Lb_h):
    def full(shape):
        return pl.BlockSpec(shape, lambda i: tuple(0 for _ in shape))

    return pl.pallas_call(
        _tc_dense_body,
        grid=(_GRID,),
        in_specs=[
            pl.BlockSpec((NC, _BLK, D), lambda i: (0, i, 0)),
            pl.BlockSpec((NC, _BLK, 16), lambda i: (0, i, 0)),
            pl.BlockSpec((_BLK, D), lambda i: (i, 0)),
            full((D, D)),
            full((1, D)),
            full((D, D)),
            full((1, D)),
            full((2 * D, D)),
            full((1, D)),
            full((2 * D, D)),
            full((1, D)),
        ],
        out_specs=pl.BlockSpec((_BLK, D), lambda i: (i, 0)),
        out_shape=jax.ShapeDtypeStruct((N, D), jnp.float32),
    )(agg_parts, deg_parts, Xs, W_z, b_z, W_h, b_h, LW_z, Lb_z, LW_h, Lb_h)


def kernel(X, edge_index, edge_weight, W_z, b_z, W_r, b_r, W_h, b_h,
           LW_z, Lb_z, LW_r, Lb_r, LW_h, Lb_h):
    row2d = edge_index[0].reshape(NW, NCH, CHUNK)
    col2d = edge_index[1].reshape(NW, NCH, CHUNK)
    ew2d = edge_weight.reshape(NW, NCH, CHUNK)
    deg_parts = _sc_deg(col2d, ew2d)
    Xs = _tc_prescale(deg_parts, X)
    agg_parts = _sc_agg(row2d, col2d, ew2d, Xs)
    return _tc_dense(
        agg_parts, deg_parts, Xs,
        W_z, b_z.reshape(1, D), W_h, b_h.reshape(1, D),
        LW_z, Lb_z.reshape(1, D), LW_h, Lb_h.reshape(1, D),
    )


# speedup vs baseline: 17.4654x; 17.4654x over previous
import jax
import jax.numpy as jnp
from jax import lax
from jax.experimental import pallas as pl
from jax.experimental.pallas import tpu as pltpu
from jax.experimental.pallas import tpu_sc as plsc

N = 10000
E = 320000
D = 128
N_PAD = 10240
NC = 2
NS = 16
NW = NC * NS
EPW = E // NW
CHUNK = 80
NCH = EPW // CHUNK
ROWS_PER_TILE = N_PAD // NS

_mesh = plsc.VectorSubcoreMesh(core_axis_name="c", subcore_axis_name="s")

_BCAST_DN = lax.GatherDimensionNumbers(
    offset_dims=(), collapsed_slice_dims=(0,), start_index_map=(0,))


def _bcast(vec, i):
    return lax.gather(
        vec, jnp.full((16, 1), i, jnp.int32), _BCAST_DN, (1,),
        mode=lax.GatherScatterMode.PROMISE_IN_BOUNDS)


def _zero_vmem(buf, rows):
    for r in range(rows):
        for db in range(D // 16):
            buf[r, pl.ds(db * 16, 16)] = jnp.zeros((16,), jnp.float32)


def _zero_shared(zbuf, shared, sub):
    _zero_vmem(zbuf, 16)
    for t in range(ROWS_PER_TILE // 16):
        pltpu.sync_copy(zbuf, shared.at[pl.ds(sub * ROWS_PER_TILE + t * 16, 16)])


def _dump_shared(zbuf, shared, out_hbm, core, sub):
    for t in range(ROWS_PER_TILE // 16):
        base = sub * ROWS_PER_TILE + t * 16
        pltpu.sync_copy(shared.at[pl.ds(base, 16)], zbuf)
        pltpu.sync_copy(zbuf, out_hbm.at[core, pl.ds(base, 16)])


def _sc_deg(col2d, ew2d):
    def body(col_hbm, ew_hbm, out_hbm, acc_sh, col_vj, ew_vj, pay_v, zbuf):
        c = lax.axis_index("c")
        s = lax.axis_index("s")
        wid = c * NS + s
        _zero_vmem(pay_v, CHUNK)
        _zero_shared(zbuf, acc_sh, s)
        plsc.subcore_barrier()

        io = lax.iota(jnp.int32, 16)
        zz = jnp.zeros((16,), jnp.float32)

        def chunk(j, carry):
            pltpu.sync_copy(col_hbm.at[pl.ds(wid * NCH + j, 1)], col_vj)
            pltpu.sync_copy(ew_hbm.at[pl.ds(wid * NCH + j, 1)], ew_vj)
            for g in range(CHUNK // 16):
                ewg = ew_vj[0, pl.ds(g * 16, 16)]
                for i in range(16):
                    pay_v[g * 16 + i, pl.ds(0, 16)] = jnp.where(io == i, ewg, zz)
            pltpu.sync_copy(pay_v, acc_sh.at[col_vj.at[0, :]], add=True)
            return carry

        lax.fori_loop(0, NCH, chunk, 0)
        plsc.subcore_barrier()
        _dump_shared(zbuf, acc_sh, out_hbm, c, s)

    k = pl.kernel(
        body,
        out_type=jax.ShapeDtypeStruct((NC, N_PAD, D), jnp.float32),
        mesh=_mesh,
        scratch_types=[
            pltpu.VMEM_SHARED((N_PAD, D), jnp.float32),
            pltpu.VMEM((1, CHUNK), jnp.int32),
            pltpu.VMEM((1, CHUNK), jnp.float32),
            pltpu.VMEM((CHUNK, D), jnp.float32),
            pltpu.VMEM((16, D), jnp.float32),
        ],
    )
    return k(col2d, ew2d)


def _sc_agg(row2d, col2d, ew2d, Xs):
    def body(row_hbm, col_hbm, ew_hbm, xs_hbm, out_hbm,
             acc_sh, row_vj, col_vj, ew_vj, rows_v, zbuf, gsem):
        c = lax.axis_index("c")
        s = lax.axis_index("s")
        wid = c * NS + s
        _zero_shared(zbuf, acc_sh, s)
        plsc.subcore_barrier()

        def chunk(j, carry):
            pltpu.sync_copy(row_hbm.at[pl.ds(wid * NCH + j, 1)], row_vj)
            pltpu.sync_copy(col_hbm.at[pl.ds(wid * NCH + j, 1)], col_vj)
            pltpu.sync_copy(ew_hbm.at[pl.ds(wid * NCH + j, 1)], ew_vj)
            pltpu.async_copy(xs_hbm.at[row_vj.at[0, :]], rows_v, gsem).wait()
            for g in range(CHUNK // 16):
                ewg = ew_vj[0, pl.ds(g * 16, 16)]
                for i in range(16):
                    e = g * 16 + i
                    sv = _bcast(ewg, i)
                    for db in range(D // 16):
                        sl = pl.ds(db * 16, 16)
                        rows_v[e, sl] = rows_v[e, sl] * sv
            pltpu.sync_copy(rows_v, acc_sh.at[col_vj.at[0, :]], add=True)
            return carry

        lax.fori_loop(0, NCH, chunk, 0)
        plsc.subcore_barrier()
        _dump_shared(zbuf, acc_sh, out_hbm, c, s)

    k = pl.kernel(
        body,
        out_type=jax.ShapeDtypeStruct((NC, N_PAD, D), jnp.float32),
        mesh=_mesh,
        scratch_types=[
            pltpu.VMEM_SHARED((N_PAD, D), jnp.float32),
            pltpu.VMEM((1, CHUNK), jnp.int32),
            pltpu.VMEM((1, CHUNK), jnp.int32),
            pltpu.VMEM((1, CHUNK), jnp.float32),
            pltpu.VMEM((CHUNK, D), jnp.float32),
            pltpu.VMEM((16, D), jnp.float32),
            pltpu.SemaphoreType.DMA,
        ],
    )
    return k(row2d, col2d, ew2d, Xs)


_BLK = 1000
_GRID = N // _BLK


def _deg_of(deg_ref):
    return (1.0 + jnp.sum(deg_ref[0, :, :16], axis=-1, keepdims=True)
            + jnp.sum(deg_ref[1, :, :16], axis=-1, keepdims=True))


def _tc_prescale_body(deg_ref, x_ref, xs_ref):
    dinv = lax.rsqrt(_deg_of(deg_ref))
    xs_ref[...] = x_ref[...] * dinv


def _tc_prescale(deg_parts, X):
    return pl.pallas_call(
        _tc_prescale_body,
        grid=(_GRID,),
        in_specs=[
            pl.BlockSpec((NC, _BLK, D), lambda i: (0, i, 0)),
            pl.BlockSpec((_BLK, D), lambda i: (i, 0)),
        ],
        out_specs=pl.BlockSpec((_BLK, D), lambda i: (i, 0)),
        out_shape=jax.ShapeDtypeStruct((N, D), jnp.float32),
    )(deg_parts, X)


def _dot(a, b):
    return lax.dot_general(
        a, b, (((1,), (0,)), ((), ())),
        precision=lax.Precision.HIGHEST,
        preferred_element_type=jnp.float32,
    )


def _tc_dense_body(agg_ref, deg_ref, xs_ref, wz_ref, bz_ref, wh_ref, bh_ref,
                   lwz_ref, lbz_ref, lwh_ref, lbh_ref, out_ref):
    dinv = lax.rsqrt(_deg_of(deg_ref))
    p = dinv * (agg_ref[0] + agg_ref[1] + xs_ref[...])
    az = lwz_ref[:D, :]
    ah = lwh_ref[:D, :]
    mz = _dot(wz_ref[...], az)
    mh = _dot(wh_ref[...], ah)
    cz = _dot(bz_ref[...], az) + lbz_ref[...]
    ch = _dot(bh_ref[...], ah) + lbh_ref[...]
    z = jax.nn.sigmoid(_dot(p, mz) + cz)
    ht = jnp.tanh(_dot(p, mh) + ch)
    out_ref[...] = (1.0 - z) * ht


def _tc_dense(agg_parts, deg_parts, Xs, W_z, b_z, W_h, b_h, LW_z, Lb_z, LW_h, Lb_h):
    def full(shape):
        return pl.BlockSpec(shape, lambda i: tuple(0 for _ in shape))

    return pl.pallas_call(
        _tc_dense_body,
        grid=(_GRID,),
        in_specs=[
            pl.BlockSpec((NC, _BLK, D), lambda i: (0, i, 0)),
            pl.BlockSpec((NC, _BLK, D), lambda i: (0, i, 0)),
            pl.BlockSpec((_BLK, D), lambda i: (i, 0)),
            full((D, D)),
            full((1, D)),
            full((D, D)),
            full((1, D)),
            full((2 * D, D)),
            full((1, D)),
            full((2 * D, D)),
            full((1, D)),
        ],
        out_specs=pl.BlockSpec((_BLK, D), lambda i: (i, 0)),
        out_shape=jax.ShapeDtypeStruct((N, D), jnp.float32),
    )(agg_parts, deg_parts, Xs, W_z, b_z, W_h, b_h, LW_z, Lb_z, LW_h, Lb_h)


def kernel(X, edge_index, edge_weight, W_z, b_z, W_r, b_r, W_h, b_h,
           LW_z, Lb_z, LW_r, Lb_r, LW_h, Lb_h):
    row2d = edge_index[0].reshape(NW * NCH, CHUNK)
    col2d = edge_index[1].reshape(NW * NCH, CHUNK)
    ew2d = edge_weight.reshape(NW * NCH, CHUNK)
    deg_parts = _sc_deg(col2d, ew2d)
    Xs = _tc_prescale(deg_parts, X)
    agg_parts = _sc_agg(row2d, col2d, ew2d, Xs)
    return _tc_dense(
        agg_parts, deg_parts, Xs,
        W_z, b_z.reshape(1, D), W_h, b_h.reshape(1, D),
        LW_z, Lb_z.reshape(1, D), LW_h, Lb_h.reshape(1, D),
    )


# speedup vs baseline: 23.1675x; 1.3265x over previous
import jax
import jax.numpy as jnp
from jax import lax
from jax.experimental import pallas as pl
from jax.experimental.pallas import tpu as pltpu
from jax.experimental.pallas import tpu_sc as plsc

N = 10000
E = 320000
D = 128
N_PAD = 10240
NC = 2
NS = 16
NW = NC * NS
CHUNK = 80
NCH = 126
E_PAD = NW * NCH * CHUNK
ROWS_PER_TILE = N_PAD // NS
NPAIR = NCH // 2

_mesh = plsc.VectorSubcoreMesh(core_axis_name="c", subcore_axis_name="s")

_BCAST_DN = lax.GatherDimensionNumbers(
    offset_dims=(), collapsed_slice_dims=(0,), start_index_map=(0,))


def _bcast(vec, i):
    return lax.gather(
        vec, jnp.full((16, 1), i, jnp.int32), _BCAST_DN, (1,),
        mode=lax.GatherScatterMode.PROMISE_IN_BOUNDS)


def _zero_vmem(buf, rows):
    for r in range(rows):
        for db in range(D // 16):
            buf[r, pl.ds(db * 16, 16)] = jnp.zeros((16,), jnp.float32)


def _zero_shared(zbuf, shared, sub):
    _zero_vmem(zbuf, 16)
    for t in range(ROWS_PER_TILE // 16):
        pltpu.sync_copy(zbuf, shared.at[pl.ds(sub * ROWS_PER_TILE + t * 16, 16)])


def _dump_shared(zbuf, shared, out_hbm, core, sub):
    for t in range(ROWS_PER_TILE // 16):
        base = sub * ROWS_PER_TILE + t * 16
        pltpu.sync_copy(shared.at[pl.ds(base, 16)], zbuf)
        pltpu.sync_copy(zbuf, out_hbm.at[core, pl.ds(base, 16)])


def _regcopy80(src, dst):
    for k in range(CHUNK // 16):
        sl = pl.ds(k * 16, 16)
        dst[0, sl] = src[0, sl]


def _sc_deg(col2d, ew2d):
    def body(col_hbm, ew_hbm, out_hbm, acc_sh,
             col_a, ew_a, cs_a, pay_a, col_b, ew_b, cs_b, pay_b,
             zbuf, isem_a, isem_b, ssem_a, ssem_b):
        c = lax.axis_index("c")
        s = lax.axis_index("s")
        wid = c * NS + s
        _zero_vmem(pay_a, CHUNK)
        _zero_vmem(pay_b, CHUNK)
        _zero_shared(zbuf, acc_sh, s)
        plsc.subcore_barrier()

        io = lax.iota(jnp.int32, 16)
        zz = jnp.zeros((16,), jnp.float32)

        def idx_issue(j, cv, ev, sem):
            base = wid * NCH + j
            pltpu.async_copy(col_hbm.at[pl.ds(base, 1)], cv, sem)
            pltpu.async_copy(ew_hbm.at[pl.ds(base, 1)], ev, sem)

        def idx_wait(cv, ev, sem):
            pltpu.make_async_copy(col_hbm.at[pl.ds(0, 1)], cv, sem).wait()
            pltpu.make_async_copy(ew_hbm.at[pl.ds(0, 1)], ev, sem).wait()

        def build(ev, pv):
            for g in range(CHUNK // 16):
                ewg = ev[0, pl.ds(g * 16, 16)]
                for i in range(16):
                    pv[g * 16 + i, pl.ds(0, 16)] = jnp.where(io == i, ewg, zz)

        def scat_issue(pv, csv, sem):
            pltpu.sync_copy(pv, acc_sh.at[csv.at[0, :]], add=True)

        def scat_wait(pv, csv, sem):
            pass

        idx_issue(0, col_a, ew_a, isem_a)
        idx_issue(1, col_b, ew_b, isem_b)
        idx_wait(col_a, ew_a, isem_a)
        idx_wait(col_b, ew_b, isem_b)

        def pair(jj, carry):
            j0 = 2 * jj

            @pl.when(jj > 0)
            def _():
                scat_wait(pay_a, cs_a, ssem_a)
                scat_wait(pay_b, cs_b, ssem_b)

            build(ew_a, pay_a)
            _regcopy80(col_a, cs_a)
            scat_issue(pay_a, cs_a, ssem_a)
            build(ew_b, pay_b)
            _regcopy80(col_b, cs_b)
            scat_issue(pay_b, cs_b, ssem_b)

            @pl.when(jj < NPAIR - 1)
            def _():
                idx_issue(j0 + 2, col_a, ew_a, isem_a)
                idx_issue(j0 + 3, col_b, ew_b, isem_b)
                idx_wait(col_a, ew_a, isem_a)
                idx_wait(col_b, ew_b, isem_b)

            return carry

        lax.fori_loop(0, NPAIR, pair, 0)
        scat_wait(pay_a, cs_a, ssem_a)
        scat_wait(pay_b, cs_b, ssem_b)
        plsc.subcore_barrier()
        _dump_shared(zbuf, acc_sh, out_hbm, c, s)

    k = pl.kernel(
        body,
        out_type=jax.ShapeDtypeStruct((NC, N_PAD, D), jnp.float32),
        mesh=_mesh,
        scratch_types=[
            pltpu.VMEM_SHARED((N_PAD, D), jnp.float32),
            pltpu.VMEM((1, CHUNK), jnp.int32),
            pltpu.VMEM((1, CHUNK), jnp.float32),
            pltpu.VMEM((1, CHUNK), jnp.int32),
            pltpu.VMEM((CHUNK, D), jnp.float32),
            pltpu.VMEM((1, CHUNK), jnp.int32),
            pltpu.VMEM((1, CHUNK), jnp.float32),
            pltpu.VMEM((1, CHUNK), jnp.int32),
            pltpu.VMEM((CHUNK, D), jnp.float32),
            pltpu.VMEM((16, D), jnp.float32),
            pltpu.SemaphoreType.DMA,
            pltpu.SemaphoreType.DMA,
            pltpu.SemaphoreType.DMA,
            pltpu.SemaphoreType.DMA,
        ],
    )
    return k(col2d, ew2d)


def _sc_agg(row2d, col2d, ew2d, Xs):
    def body(row_hbm, col_hbm, ew_hbm, xs_hbm, out_hbm, acc_sh,
             row_a, col_a, ew_a, cs_a, rows_a,
             row_b, col_b, ew_b, cs_b, rows_b,
             zbuf, isem_a, isem_b, gsem_a, gsem_b, ssem_a, ssem_b):
        c = lax.axis_index("c")
        s = lax.axis_index("s")
        wid = c * NS + s
        _zero_shared(zbuf, acc_sh, s)
        plsc.subcore_barrier()

        def idx_issue(j, rv, cv, ev, sem):
            base = wid * NCH + j
            pltpu.async_copy(row_hbm.at[pl.ds(base, 1)], rv, sem)
            pltpu.async_copy(col_hbm.at[pl.ds(base, 1)], cv, sem)
            pltpu.async_copy(ew_hbm.at[pl.ds(base, 1)], ev, sem)

        def idx_wait(rv, cv, ev, sem):
            pltpu.make_async_copy(row_hbm.at[pl.ds(0, 1)], rv, sem).wait()
            pltpu.make_async_copy(col_hbm.at[pl.ds(0, 1)], cv, sem).wait()
            pltpu.make_async_copy(ew_hbm.at[pl.ds(0, 1)], ev, sem).wait()

        def gat_issue(rv, dst, sem):
            pltpu.async_copy(xs_hbm.at[rv.at[0, :]], dst, sem)

        def gat_wait(rv, dst, sem):
            pltpu.make_async_copy(xs_hbm.at[rv.at[0, :]], dst, sem).wait()

        def scale(ev, rowsv):
            for g in range(CHUNK // 16):
                ewg = ev[0, pl.ds(g * 16, 16)]
                for i in range(16):
                    e = g * 16 + i
                    sv = _bcast(ewg, i)
                    for db in range(D // 16):
                        sl = pl.ds(db * 16, 16)
                        rowsv[e, sl] = rowsv[e, sl] * sv

        def scat_issue(rowsv, csv, sem):
            pltpu.sync_copy(rowsv, acc_sh.at[csv.at[0, :]], add=True)

        def scat_wait(rowsv, csv, sem):
            pass

        idx_issue(0, row_a, col_a, ew_a, isem_a)
        idx_issue(1, row_b, col_b, ew_b, isem_b)
        idx_wait(row_a, col_a, ew_a, isem_a)
        gat_issue(row_a, rows_a, gsem_a)
        idx_wait(row_b, col_b, ew_b, isem_b)
        gat_issue(row_b, rows_b, gsem_b)

        def pair(jj, carry):
            j0 = 2 * jj
            last = jj >= NPAIR - 1
            gat_wait(row_a, rows_a, gsem_a)
            scale(ew_a, rows_a)
            _regcopy80(col_a, cs_a)
            scat_issue(rows_a, cs_a, ssem_a)

            @pl.when(jnp.logical_not(last))
            def _():
                idx_issue(j0 + 2, row_a, col_a, ew_a, isem_a)

            gat_wait(row_b, rows_b, gsem_b)
            scale(ew_b, rows_b)
            _regcopy80(col_b, cs_b)
            scat_issue(rows_b, cs_b, ssem_b)

            @pl.when(jnp.logical_not(last))
            def _():
                idx_issue(j0 + 3, row_b, col_b, ew_b, isem_b)

            scat_wait(rows_a, cs_a, ssem_a)
            scat_wait(rows_b, cs_b, ssem_b)

            @pl.when(jnp.logical_not(last))
            def _():
                idx_wait(row_a, col_a, ew_a, isem_a)
                gat_issue(row_a, rows_a, gsem_a)
                idx_wait(row_b, col_b, ew_b, isem_b)
                gat_issue(row_b, rows_b, gsem_b)

            return carry

        lax.fori_loop(0, NPAIR, pair, 0)
        plsc.subcore_barrier()
        _dump_shared(zbuf, acc_sh, out_hbm, c, s)

    k = pl.kernel(
        body,
        out_type=jax.ShapeDtypeStruct((NC, N_PAD, D), jnp.float32),
        mesh=_mesh,
        scratch_types=[
            pltpu.VMEM_SHARED((N_PAD, D), jnp.float32),
            pltpu.VMEM((1, CHUNK), jnp.int32),
            pltpu.VMEM((1, CHUNK), jnp.int32),
            pltpu.VMEM((1, CHUNK), jnp.float32),
            pltpu.VMEM((1, CHUNK), jnp.int32),
            pltpu.VMEM((CHUNK, D), jnp.float32),
            pltpu.VMEM((1, CHUNK), jnp.int32),
            pltpu.VMEM((1, CHUNK), jnp.int32),
            pltpu.VMEM((1, CHUNK), jnp.float32),
            pltpu.VMEM((1, CHUNK), jnp.int32),
            pltpu.VMEM((CHUNK, D), jnp.float32),
            pltpu.VMEM((16, D), jnp.float32),
            pltpu.SemaphoreType.DMA,
            pltpu.SemaphoreType.DMA,
            pltpu.SemaphoreType.DMA,
            pltpu.SemaphoreType.DMA,
            pltpu.SemaphoreType.DMA,
            pltpu.SemaphoreType.DMA,
        ],
    )
    return k(row2d, col2d, ew2d, Xs)


_BLK = 1000
_GRID = N // _BLK


def _deg_of(deg_ref):
    return (1.0 + jnp.sum(deg_ref[0, :, :16], axis=-1, keepdims=True)
            + jnp.sum(deg_ref[1, :, :16], axis=-1, keepdims=True))


def _tc_prescale_body(deg_ref, x_ref, xs_ref):
    dinv = lax.rsqrt(_deg_of(deg_ref))
    xs_ref[...] = x_ref[...] * dinv


def _tc_prescale(deg_parts, X):
    return pl.pallas_call(
        _tc_prescale_body,
        grid=(_GRID,),
        in_specs=[
            pl.BlockSpec((NC, _BLK, D), lambda i: (0, i, 0)),
            pl.BlockSpec((_BLK, D), lambda i: (i, 0)),
        ],
        out_specs=pl.BlockSpec((_BLK, D), lambda i: (i, 0)),
        out_shape=jax.ShapeDtypeStruct((N, D), jnp.float32),
    )(deg_parts, X)


def _dot(a, b):
    return lax.dot_general(
        a, b, (((1,), (0,)), ((), ())),
        precision=lax.Precision.HIGHEST,
        preferred_element_type=jnp.float32,
    )


def _tc_dense_body(agg_ref, deg_ref, xs_ref, wz_ref, bz_ref, wh_ref, bh_ref,
                   lwz_ref, lbz_ref, lwh_ref, lbh_ref, out_ref):
    dinv = lax.rsqrt(_deg_of(deg_ref))
    p = dinv * (agg_ref[0] + agg_ref[1] + xs_ref[...])
    az = lwz_ref[:D, :]
    ah = lwh_ref[:D, :]
    mz = _dot(wz_ref[...], az)
    mh = _dot(wh_ref[...], ah)
    cz = _dot(bz_ref[...], az) + lbz_ref[...]
    ch = _dot(bh_ref[...], ah) + lbh_ref[...]
    z = jax.nn.sigmoid(_dot(p, mz) + cz)
    ht = jnp.tanh(_dot(p, mh) + ch)
    out_ref[...] = (1.0 - z) * ht


def _tc_dense(agg_parts, deg_parts, Xs, W_z, b_z, W_h, b_h, LW_z, Lb_z, LW_h, Lb_h):
    def full(shape):
        return pl.BlockSpec(shape, lambda i: tuple(0 for _ in shape))

    return pl.pallas_call(
        _tc_dense_body,
        grid=(_GRID,),
        in_specs=[
            pl.BlockSpec((NC, _BLK, D), lambda i: (0, i, 0)),
            pl.BlockSpec((NC, _BLK, D), lambda i: (0, i, 0)),
            pl.BlockSpec((_BLK, D), lambda i: (i, 0)),
            full((D, D)),
            full((1, D)),
            full((D, D)),
            full((1, D)),
            full((2 * D, D)),
            full((1, D)),
            full((2 * D, D)),
            full((1, D)),
        ],
        out_specs=pl.BlockSpec((_BLK, D), lambda i: (i, 0)),
        out_shape=jax.ShapeDtypeStruct((N, D), jnp.float32),
    )(agg_parts, deg_parts, Xs, W_z, b_z, W_h, b_h, LW_z, Lb_z, LW_h, Lb_h)


def kernel(X, edge_index, edge_weight, W_z, b_z, W_r, b_r, W_h, b_h,
           LW_z, Lb_z, LW_r, Lb_r, LW_h, Lb_h):
    pad = E_PAD - E
    row_p = jnp.concatenate([edge_index[0], jnp.zeros((pad,), edge_index.dtype)])
    col_p = jnp.concatenate([edge_index[1], jnp.zeros((pad,), edge_index.dtype)])
    ew_p = jnp.concatenate([edge_weight, jnp.zeros((pad,), edge_weight.dtype)])
    row2d = row_p.reshape(NW * NCH, CHUNK)
    col2d = col_p.reshape(NW * NCH, CHUNK)
    ew2d = ew_p.reshape(NW * NCH, CHUNK)
    deg_parts = _sc_deg(col2d, ew2d)
    Xs = _tc_prescale(deg_parts, X)
    agg_parts = _sc_agg(row2d, col2d, ew2d, Xs)
    return _tc_dense(
        agg_parts, deg_parts, Xs,
        W_z, b_z.reshape(1, D), W_h, b_h.reshape(1, D),
        LW_z, Lb_z.reshape(1, D), LW_h, Lb_h.reshape(1, D),
    )


# speedup vs baseline: 23.7784x; 1.0264x over previous
import jax
import jax.numpy as jnp
from jax import lax
from jax.experimental import pallas as pl
from jax.experimental.pallas import tpu as pltpu
from jax.experimental.pallas import tpu_sc as plsc

N = 10000
E = 320000
D = 128
N_PAD = 10240
NC = 2
NS = 16
NW = NC * NS
CHUNK = 80
NCH = 126
E_PAD = NW * NCH * CHUNK
ROWS_PER_TILE = N_PAD // NS
NPAIR = NCH // 2

_mesh = plsc.VectorSubcoreMesh(core_axis_name="c", subcore_axis_name="s")

_BCAST_DN = lax.GatherDimensionNumbers(
    offset_dims=(), collapsed_slice_dims=(0,), start_index_map=(0,))


def _bcast(vec, i):
    return lax.gather(
        vec, jnp.full((16, 1), i, jnp.int32), _BCAST_DN, (1,),
        mode=lax.GatherScatterMode.PROMISE_IN_BOUNDS)


def _zero_vmem(buf, rows):
    for r in range(rows):
        for db in range(D // 16):
            buf[r, pl.ds(db * 16, 16)] = jnp.zeros((16,), jnp.float32)


def _zero_shared(zbuf, shared, sub):
    _zero_vmem(zbuf, 16)
    for t in range(ROWS_PER_TILE // 16):
        pltpu.sync_copy(zbuf, shared.at[pl.ds(sub * ROWS_PER_TILE + t * 16, 16)])


def _dump_shared(zbuf, shared, out_hbm, core, sub):
    for t in range(ROWS_PER_TILE // 16):
        base = sub * ROWS_PER_TILE + t * 16
        pltpu.sync_copy(shared.at[pl.ds(base, 16)], zbuf)
        pltpu.sync_copy(zbuf, out_hbm.at[core, pl.ds(base, 16)])


def _regcopy80(src, dst):
    for k in range(CHUNK // 16):
        sl = pl.ds(k * 16, 16)
        dst[0, sl] = src[0, sl]


def _sc_deg(col2d, ew2d):
    def body(col_hbm, ew_hbm, out_hbm, acc_sh,
             col_a, ew_a, cs_a, pay_a, col_b, ew_b, cs_b, pay_b,
             zbuf, isem_a, isem_b, ssem_a, ssem_b):
        c = lax.axis_index("c")
        s = lax.axis_index("s")
        wid = c * NS + s
        _zero_vmem(pay_a, CHUNK)
        _zero_vmem(pay_b, CHUNK)
        _zero_shared(zbuf, acc_sh, s)
        plsc.subcore_barrier()

        io = lax.iota(jnp.int32, 16)
        zz = jnp.zeros((16,), jnp.float32)

        def idx_issue(j, cv, ev, sem):
            base = wid * NCH + j
            pltpu.async_copy(col_hbm.at[pl.ds(base, 1)], cv, sem)
            pltpu.async_copy(ew_hbm.at[pl.ds(base, 1)], ev, sem)

        def idx_wait(cv, ev, sem):
            pltpu.make_async_copy(col_hbm.at[pl.ds(0, 1)], cv, sem).wait()
            pltpu.make_async_copy(ew_hbm.at[pl.ds(0, 1)], ev, sem).wait()

        def build(ev, pv):
            for g in range(CHUNK // 16):
                ewg = ev[0, pl.ds(g * 16, 16)]
                for i in range(16):
                    pv[g * 16 + i, pl.ds(0, 16)] = jnp.where(io == i, ewg, zz)

        def scat_issue(pv, csv, sem):
            pltpu.async_copy(pv, acc_sh.at[csv.at[0, :]], sem, add=True)

        def scat_wait(pv, csv, sem):
            pltpu.make_async_copy(pv, acc_sh.at[csv.at[0, :]], sem).wait()

        idx_issue(0, col_a, ew_a, isem_a)
        idx_issue(1, col_b, ew_b, isem_b)
        idx_wait(col_a, ew_a, isem_a)
        idx_wait(col_b, ew_b, isem_b)

        def pair(jj, carry):
            j0 = 2 * jj
            build(ew_a, pay_a)
            _regcopy80(col_a, cs_a)
            scat_issue(pay_a, cs_a, ssem_a)
            build(ew_b, pay_b)
            _regcopy80(col_b, cs_b)
            scat_wait(pay_a, cs_a, ssem_a)
            pltpu.sync_copy(pay_b, acc_sh.at[cs_b.at[0, :]], add=True)

            @pl.when(jj < NPAIR - 1)
            def _():
                idx_issue(j0 + 2, col_a, ew_a, isem_a)
                idx_issue(j0 + 3, col_b, ew_b, isem_b)
                idx_wait(col_a, ew_a, isem_a)
                idx_wait(col_b, ew_b, isem_b)

            return carry

        lax.fori_loop(0, NPAIR, pair, 0)
        plsc.subcore_barrier()
        _dump_shared(zbuf, acc_sh, out_hbm, c, s)

    k = pl.kernel(
        body,
        out_type=jax.ShapeDtypeStruct((NC, N_PAD, D), jnp.float32),
        mesh=_mesh,
        scratch_types=[
            pltpu.VMEM_SHARED((N_PAD, D), jnp.float32),
            pltpu.VMEM((1, CHUNK), jnp.int32),
            pltpu.VMEM((1, CHUNK), jnp.float32),
            pltpu.VMEM((1, CHUNK), jnp.int32),
            pltpu.VMEM((CHUNK, D), jnp.float32),
            pltpu.VMEM((1, CHUNK), jnp.int32),
            pltpu.VMEM((1, CHUNK), jnp.float32),
            pltpu.VMEM((1, CHUNK), jnp.int32),
            pltpu.VMEM((CHUNK, D), jnp.float32),
            pltpu.VMEM((16, D), jnp.float32),
            pltpu.SemaphoreType.DMA,
            pltpu.SemaphoreType.DMA,
            pltpu.SemaphoreType.DMA,
            pltpu.SemaphoreType.DMA,
        ],
    )
    return k(col2d, ew2d)


def _sc_agg(row2d, col2d, ew2d, Xs):
    def body(row_hbm, col_hbm, ew_hbm, xs_hbm, out_hbm, acc_sh,
             row_a, col_a, ew_a, cs_a, rows_a,
             row_b, col_b, ew_b, cs_b, rows_b,
             zbuf, isem_a, isem_b, gsem_a, gsem_b, ssem_a, ssem_b):
        c = lax.axis_index("c")
        s = lax.axis_index("s")
        wid = c * NS + s
        _zero_shared(zbuf, acc_sh, s)
        plsc.subcore_barrier()

        def idx_issue(j, rv, cv, ev, sem):
            base = wid * NCH + j
            pltpu.async_copy(row_hbm.at[pl.ds(base, 1)], rv, sem)
            pltpu.async_copy(col_hbm.at[pl.ds(base, 1)], cv, sem)
            pltpu.async_copy(ew_hbm.at[pl.ds(base, 1)], ev, sem)

        def idx_wait(rv, cv, ev, sem):
            pltpu.make_async_copy(row_hbm.at[pl.ds(0, 1)], rv, sem).wait()
            pltpu.make_async_copy(col_hbm.at[pl.ds(0, 1)], cv, sem).wait()
            pltpu.make_async_copy(ew_hbm.at[pl.ds(0, 1)], ev, sem).wait()

        def gat_issue(rv, dst, sem):
            pltpu.async_copy(xs_hbm.at[rv.at[0, :]], dst, sem)

        def gat_wait(rv, dst, sem):
            pltpu.make_async_copy(xs_hbm.at[rv.at[0, :]], dst, sem).wait()

        def scale(ev, rowsv):
            for g in range(CHUNK // 16):
                ewg = ev[0, pl.ds(g * 16, 16)]
                for i in range(16):
                    e = g * 16 + i
                    sv = _bcast(ewg, i)
                    for db in range(D // 16):
                        sl = pl.ds(db * 16, 16)
                        rowsv[e, sl] = rowsv[e, sl] * sv

        def scat_issue(rowsv, csv, sem):
            pltpu.async_copy(rowsv, acc_sh.at[csv.at[0, :]], sem, add=True)

        def scat_wait(rowsv, csv, sem):
            pltpu.make_async_copy(rowsv, acc_sh.at[csv.at[0, :]], sem).wait()

        idx_issue(0, row_a, col_a, ew_a, isem_a)
        idx_issue(1, row_b, col_b, ew_b, isem_b)
        idx_wait(row_a, col_a, ew_a, isem_a)
        gat_issue(row_a, rows_a, gsem_a)
        idx_wait(row_b, col_b, ew_b, isem_b)
        gat_issue(row_b, rows_b, gsem_b)

        def pair(jj, carry):
            j0 = 2 * jj
            last = jj >= NPAIR - 1
            gat_wait(row_a, rows_a, gsem_a)
            scale(ew_a, rows_a)
            _regcopy80(col_a, cs_a)
            scat_issue(rows_a, cs_a, ssem_a)

            @pl.when(jnp.logical_not(last))
            def _():
                idx_issue(j0 + 2, row_a, col_a, ew_a, isem_a)

            gat_wait(row_b, rows_b, gsem_b)
            scale(ew_b, rows_b)
            _regcopy80(col_b, cs_b)
            scat_wait(rows_a, cs_a, ssem_a)
            pltpu.sync_copy(rows_b, acc_sh.at[cs_b.at[0, :]], add=True)

            @pl.when(jnp.logical_not(last))
            def _():
                idx_issue(j0 + 3, row_b, col_b, ew_b, isem_b)
                idx_wait(row_a, col_a, ew_a, isem_a)
                gat_issue(row_a, rows_a, gsem_a)
                idx_wait(row_b, col_b, ew_b, isem_b)
                gat_issue(row_b, rows_b, gsem_b)

            return carry

        lax.fori_loop(0, NPAIR, pair, 0)
        plsc.subcore_barrier()
        _dump_shared(zbuf, acc_sh, out_hbm, c, s)

    k = pl.kernel(
        body,
        out_type=jax.ShapeDtypeStruct((NC, N_PAD, D), jnp.float32),
        mesh=_mesh,
        scratch_types=[
            pltpu.VMEM_SHARED((N_PAD, D), jnp.float32),
            pltpu.VMEM((1, CHUNK), jnp.int32),
            pltpu.VMEM((1, CHUNK), jnp.int32),
            pltpu.VMEM((1, CHUNK), jnp.float32),
            pltpu.VMEM((1, CHUNK), jnp.int32),
            pltpu.VMEM((CHUNK, D), jnp.float32),
            pltpu.VMEM((1, CHUNK), jnp.int32),
            pltpu.VMEM((1, CHUNK), jnp.int32),
            pltpu.VMEM((1, CHUNK), jnp.float32),
            pltpu.VMEM((1, CHUNK), jnp.int32),
            pltpu.VMEM((CHUNK, D), jnp.float32),
            pltpu.VMEM((16, D), jnp.float32),
            pltpu.SemaphoreType.DMA,
            pltpu.SemaphoreType.DMA,
            pltpu.SemaphoreType.DMA,
            pltpu.SemaphoreType.DMA,
            pltpu.SemaphoreType.DMA,
            pltpu.SemaphoreType.DMA,
        ],
    )
    return k(row2d, col2d, ew2d, Xs)


_BLK = 1000
_GRID = N // _BLK


def _deg_of(deg_ref):
    return (1.0 + jnp.sum(deg_ref[0, :, :16], axis=-1, keepdims=True)
            + jnp.sum(deg_ref[1, :, :16], axis=-1, keepdims=True))


def _tc_prescale_body(deg_ref, x_ref, xs_ref):
    dinv = lax.rsqrt(_deg_of(deg_ref))
    xs_ref[...] = x_ref[...] * dinv


def _tc_prescale(deg_parts, X):
    return pl.pallas_call(
        _tc_prescale_body,
        grid=(_GRID,),
        in_specs=[
            pl.BlockSpec((NC, _BLK, D), lambda i: (0, i, 0)),
            pl.BlockSpec((_BLK, D), lambda i: (i, 0)),
        ],
        out_specs=pl.BlockSpec((_BLK, D), lambda i: (i, 0)),
        out_shape=jax.ShapeDtypeStruct((N, D), jnp.float32),
    )(deg_parts, X)


def _dot(a, b):
    return lax.dot_general(
        a, b, (((1,), (0,)), ((), ())),
        precision=lax.Precision.HIGHEST,
        preferred_element_type=jnp.float32,
    )


def _tc_dense_body(agg_ref, deg_ref, xs_ref, wz_ref, bz_ref, wh_ref, bh_ref,
                   lwz_ref, lbz_ref, lwh_ref, lbh_ref, out_ref):
    dinv = lax.rsqrt(_deg_of(deg_ref))
    p = dinv * (agg_ref[0] + agg_ref[1] + xs_ref[...])
    az = lwz_ref[:D, :]
    ah = lwh_ref[:D, :]
    mz = _dot(wz_ref[...], az)
    mh = _dot(wh_ref[...], ah)
    cz = _dot(bz_ref[...], az) + lbz_ref[...]
    ch = _dot(bh_ref[...], ah) + lbh_ref[...]
    z = jax.nn.sigmoid(_dot(p, mz) + cz)
    ht = jnp.tanh(_dot(p, mh) + ch)
    out_ref[...] = (1.0 - z) * ht


def _tc_dense(agg_parts, deg_parts, Xs, W_z, b_z, W_h, b_h, LW_z, Lb_z, LW_h, Lb_h):
    def full(shape):
        return pl.BlockSpec(shape, lambda i: tuple(0 for _ in shape))

    return pl.pallas_call(
        _tc_dense_body,
        grid=(_GRID,),
        in_specs=[
            pl.BlockSpec((NC, _BLK, D), lambda i: (0, i, 0)),
            pl.BlockSpec((NC, _BLK, D), lambda i: (0, i, 0)),
            pl.BlockSpec((_BLK, D), lambda i: (i, 0)),
            full((D, D)),
            full((1, D)),
            full((D, D)),
            full((1, D)),
            full((2 * D, D)),
            full((1, D)),
            full((2 * D, D)),
            full((1, D)),
        ],
        out_specs=pl.BlockSpec((_BLK, D), lambda i: (i, 0)),
        out_shape=jax.ShapeDtypeStruct((N, D), jnp.float32),
    )(agg_parts, deg_parts, Xs, W_z, b_z, W_h, b_h, LW_z, Lb_z, LW_h, Lb_h)


def kernel(X, edge_index, edge_weight, W_z, b_z, W_r, b_r, W_h, b_h,
           LW_z, Lb_z, LW_r, Lb_r, LW_h, Lb_h):
    pad = E_PAD - E
    row_p = jnp.concatenate([edge_index[0], jnp.zeros((pad,), edge_index.dtype)])
    col_p = jnp.concatenate([edge_index[1], jnp.zeros((pad,), edge_index.dtype)])
    ew_p = jnp.concatenate([edge_weight, jnp.zeros((pad,), edge_weight.dtype)])
    row2d = row_p.reshape(NW * NCH, CHUNK)
    col2d = col_p.reshape(NW * NCH, CHUNK)
    ew2d = ew_p.reshape(NW * NCH, CHUNK)
    deg_parts = _sc_deg(col2d, ew2d)
    Xs = _tc_prescale(deg_parts, X)
    agg_parts = _sc_agg(row2d, col2d, ew2d, Xs)
    return _tc_dense(
        agg_parts, deg_parts, Xs,
        W_z, b_z.reshape(1, D), W_h, b_h.reshape(1, D),
        LW_z, Lb_z.reshape(1, D), LW_h, Lb_h.reshape(1, D),
    )


# speedup vs baseline: 25.7342x; 1.0823x over previous
import jax
import jax.numpy as jnp
from jax import lax
from jax.experimental import pallas as pl
from jax.experimental.pallas import tpu as pltpu
from jax.experimental.pallas import tpu_sc as plsc

N = 10000
E = 320000
D = 128
N_PAD = 10240
NC = 2
NS = 16
NW = NC * NS
CHUNK = 80
NCH = 126
E_PAD = NW * NCH * CHUNK
ROWS_PER_TILE = N_PAD // NS
NPAIR = NCH // 2
NCH_C0 = 158
NCH_C1 = 94

_mesh = plsc.VectorSubcoreMesh(core_axis_name="c", subcore_axis_name="s")

_BCAST_DN = lax.GatherDimensionNumbers(
    offset_dims=(), collapsed_slice_dims=(0,), start_index_map=(0,))


def _bcast(vec, i):
    return lax.gather(
        vec, jnp.full((16, 1), i, jnp.int32), _BCAST_DN, (1,),
        mode=lax.GatherScatterMode.PROMISE_IN_BOUNDS)


def _zero_vmem(buf, rows):
    for r in range(rows):
        for db in range(D // 16):
            buf[r, pl.ds(db * 16, 16)] = jnp.zeros((16,), jnp.float32)


def _zero_shared(zbuf, shared, sub):
    _zero_vmem(zbuf, 16)
    for t in range(ROWS_PER_TILE // 16):
        pltpu.sync_copy(zbuf, shared.at[pl.ds(sub * ROWS_PER_TILE + t * 16, 16)])


def _dump_shared(zbuf, shared, out_hbm, core, sub):
    for t in range(ROWS_PER_TILE // 16):
        base = sub * ROWS_PER_TILE + t * 16
        pltpu.sync_copy(shared.at[pl.ds(base, 16)], zbuf)
        pltpu.sync_copy(zbuf, out_hbm.at[core, pl.ds(base, 16)])


def _regcopy80(src, dst):
    for k in range(CHUNK // 16):
        sl = pl.ds(k * 16, 16)
        dst[0, sl] = src[0, sl]


def _sc_deg(col2d, ew2d):
    def body(col_hbm, ew_hbm, out_hbm, acc_sh,
             col_a, ew_a, cs_a, pay_a, col_b, ew_b, cs_b, pay_b,
             zbuf, isem_a, isem_b, ssem_a, ssem_b):
        c = lax.axis_index("c")
        s = lax.axis_index("s")
        wid = c * NS + s
        _zero_vmem(pay_a, CHUNK)
        _zero_vmem(pay_b, CHUNK)
        _zero_shared(zbuf, acc_sh, s)
        plsc.subcore_barrier()

        io = lax.iota(jnp.int32, 16)
        zz = jnp.zeros((16,), jnp.float32)

        def idx_issue(j, cv, ev, sem):
            base = wid * NCH + j
            pltpu.async_copy(col_hbm.at[pl.ds(base, 1)], cv, sem)
            pltpu.async_copy(ew_hbm.at[pl.ds(base, 1)], ev, sem)

        def idx_wait(cv, ev, sem):
            pltpu.make_async_copy(col_hbm.at[pl.ds(0, 1)], cv, sem).wait()
            pltpu.make_async_copy(ew_hbm.at[pl.ds(0, 1)], ev, sem).wait()

        def build(ev, pv):
            for g in range(CHUNK // 16):
                ewg = ev[0, pl.ds(g * 16, 16)]
                for i in range(16):
                    pv[g * 16 + i, pl.ds(0, 16)] = jnp.where(io == i, ewg, zz)

        def scat_issue(pv, csv, sem):
            pltpu.async_copy(pv, acc_sh.at[csv.at[0, :]], sem, add=True)

        def scat_wait(pv, csv, sem):
            pltpu.make_async_copy(pv, acc_sh.at[csv.at[0, :]], sem).wait()

        idx_issue(0, col_a, ew_a, isem_a)
        idx_issue(1, col_b, ew_b, isem_b)
        idx_wait(col_a, ew_a, isem_a)
        idx_wait(col_b, ew_b, isem_b)

        def pair(jj, carry):
            j0 = 2 * jj
            build(ew_a, pay_a)
            _regcopy80(col_a, cs_a)
            scat_issue(pay_a, cs_a, ssem_a)
            build(ew_b, pay_b)
            _regcopy80(col_b, cs_b)
            scat_wait(pay_a, cs_a, ssem_a)
            pltpu.sync_copy(pay_b, acc_sh.at[cs_b.at[0, :]], add=True)

            @pl.when(jj < NPAIR - 1)
            def _():
                idx_issue(j0 + 2, col_a, ew_a, isem_a)
                idx_issue(j0 + 3, col_b, ew_b, isem_b)
                idx_wait(col_a, ew_a, isem_a)
                idx_wait(col_b, ew_b, isem_b)

            return carry

        lax.fori_loop(0, NPAIR, pair, 0)
        plsc.subcore_barrier()
        _dump_shared(zbuf, acc_sh, out_hbm, c, s)

    k = pl.kernel(
        body,
        out_type=jax.ShapeDtypeStruct((NC, N_PAD, D), jnp.float32),
        mesh=_mesh,
        scratch_types=[
            pltpu.VMEM_SHARED((N_PAD, D), jnp.float32),
            pltpu.VMEM((1, CHUNK), jnp.int32),
            pltpu.VMEM((1, CHUNK), jnp.float32),
            pltpu.VMEM((1, CHUNK), jnp.int32),
            pltpu.VMEM((CHUNK, D), jnp.float32),
            pltpu.VMEM((1, CHUNK), jnp.int32),
            pltpu.VMEM((1, CHUNK), jnp.float32),
            pltpu.VMEM((1, CHUNK), jnp.int32),
            pltpu.VMEM((CHUNK, D), jnp.float32),
            pltpu.VMEM((16, D), jnp.float32),
            pltpu.SemaphoreType.DMA,
            pltpu.SemaphoreType.DMA,
            pltpu.SemaphoreType.DMA,
            pltpu.SemaphoreType.DMA,
        ],
    )
    return k(col2d, ew2d)


def _sc_agg(row2d, col2d, ew2d, Xs):
    def body(row_hbm, col_hbm, ew_hbm, xs_hbm, out_hbm, acc_sh,
             row_a, col_a, ew_a, cs_a, rows_a,
             row_b, col_b, ew_b, cs_b, rows_b,
             zbuf, isem_a, isem_b, gsem_a, gsem_b, ssem_a, ssem_b):
        c = lax.axis_index("c")
        s = lax.axis_index("s")
        _zero_shared(zbuf, acc_sh, s)
        plsc.subcore_barrier()
        slab = jnp.where(c == 0, s * NCH_C0, NS * NCH_C0 + s * NCH_C1)
        npair = jnp.where(c == 0, NCH_C0 // 2, NCH_C1 // 2)

        def idx_issue(j, rv, cv, ev, sem):
            base = slab + j
            pltpu.async_copy(row_hbm.at[pl.ds(base, 1)], rv, sem)
            pltpu.async_copy(col_hbm.at[pl.ds(base, 1)], cv, sem)
            pltpu.async_copy(ew_hbm.at[pl.ds(base, 1)], ev, sem)

        def idx_wait(rv, cv, ev, sem):
            pltpu.make_async_copy(row_hbm.at[pl.ds(0, 1)], rv, sem).wait()
            pltpu.make_async_copy(col_hbm.at[pl.ds(0, 1)], cv, sem).wait()
            pltpu.make_async_copy(ew_hbm.at[pl.ds(0, 1)], ev, sem).wait()

        def gat_issue(rv, dst, sem):
            pltpu.async_copy(xs_hbm.at[rv.at[0, :]], dst, sem)

        def gat_wait(rv, dst, sem):
            pltpu.make_async_copy(xs_hbm.at[rv.at[0, :]], dst, sem).wait()

        def scale(ev, rowsv):
            for g in range(CHUNK // 16):
                ewg = ev[0, pl.ds(g * 16, 16)]
                for i in range(16):
                    e = g * 16 + i
                    sv = _bcast(ewg, i)
                    for db in range(D // 16):
                        sl = pl.ds(db * 16, 16)
                        rowsv[e, sl] = rowsv[e, sl] * sv

        def scat_issue(rowsv, csv, sem):
            pltpu.async_copy(rowsv, acc_sh.at[csv.at[0, :]], sem, add=True)

        def scat_wait(rowsv, csv, sem):
            pltpu.make_async_copy(rowsv, acc_sh.at[csv.at[0, :]], sem).wait()

        idx_issue(0, row_a, col_a, ew_a, isem_a)
        idx_issue(1, row_b, col_b, ew_b, isem_b)
        idx_wait(row_a, col_a, ew_a, isem_a)
        gat_issue(row_a, rows_a, gsem_a)
        idx_wait(row_b, col_b, ew_b, isem_b)
        gat_issue(row_b, rows_b, gsem_b)

        def pair(jj, carry):
            j0 = 2 * jj
            last = jj >= npair - 1
            gat_wait(row_a, rows_a, gsem_a)
            scale(ew_a, rows_a)
            _regcopy80(col_a, cs_a)
            scat_issue(rows_a, cs_a, ssem_a)

            @pl.when(jnp.logical_not(last))
            def _():
                idx_issue(j0 + 2, row_a, col_a, ew_a, isem_a)

            gat_wait(row_b, rows_b, gsem_b)
            scale(ew_b, rows_b)
            _regcopy80(col_b, cs_b)
            scat_wait(rows_a, cs_a, ssem_a)
            pltpu.sync_copy(rows_b, acc_sh.at[cs_b.at[0, :]], add=True)

            @pl.when(jnp.logical_not(last))
            def _():
                idx_issue(j0 + 3, row_b, col_b, ew_b, isem_b)
                idx_wait(row_a, col_a, ew_a, isem_a)
                gat_issue(row_a, rows_a, gsem_a)
                idx_wait(row_b, col_b, ew_b, isem_b)
                gat_issue(row_b, rows_b, gsem_b)

            return carry

        lax.fori_loop(0, npair, pair, 0)
        plsc.subcore_barrier()
        _dump_shared(zbuf, acc_sh, out_hbm, c, s)

    k = pl.kernel(
        body,
        out_type=jax.ShapeDtypeStruct((NC, N_PAD, D), jnp.float32),
        mesh=_mesh,
        scratch_types=[
            pltpu.VMEM_SHARED((N_PAD, D), jnp.float32),
            pltpu.VMEM((1, CHUNK), jnp.int32),
            pltpu.VMEM((1, CHUNK), jnp.int32),
            pltpu.VMEM((1, CHUNK), jnp.float32),
            pltpu.VMEM((1, CHUNK), jnp.int32),
            pltpu.VMEM((CHUNK, D), jnp.float32),
            pltpu.VMEM((1, CHUNK), jnp.int32),
            pltpu.VMEM((1, CHUNK), jnp.int32),
            pltpu.VMEM((1, CHUNK), jnp.float32),
            pltpu.VMEM((1, CHUNK), jnp.int32),
            pltpu.VMEM((CHUNK, D), jnp.float32),
            pltpu.VMEM((16, D), jnp.float32),
            pltpu.SemaphoreType.DMA,
            pltpu.SemaphoreType.DMA,
            pltpu.SemaphoreType.DMA,
            pltpu.SemaphoreType.DMA,
            pltpu.SemaphoreType.DMA,
            pltpu.SemaphoreType.DMA,
        ],
    )
    return k(row2d, col2d, ew2d, Xs)


_BLK = 1000
_GRID = N // _BLK


def _deg_of(deg_ref):
    return (1.0 + jnp.sum(deg_ref[0, :, :16], axis=-1, keepdims=True)
            + jnp.sum(deg_ref[1, :, :16], axis=-1, keepdims=True))


def _tc_prescale_body(deg_ref, x_ref, xs_ref):
    dinv = lax.rsqrt(_deg_of(deg_ref))
    xs_ref[...] = x_ref[...] * dinv


def _tc_prescale(deg_parts, X):
    return pl.pallas_call(
        _tc_prescale_body,
        grid=(_GRID,),
        in_specs=[
            pl.BlockSpec((NC, _BLK, D), lambda i: (0, i, 0)),
            pl.BlockSpec((_BLK, D), lambda i: (i, 0)),
        ],
        out_specs=pl.BlockSpec((_BLK, D), lambda i: (i, 0)),
        out_shape=jax.ShapeDtypeStruct((N, D), jnp.float32),
    )(deg_parts, X)


def _dot(a, b):
    return lax.dot_general(
        a, b, (((1,), (0,)), ((), ())),
        precision=lax.Precision.HIGHEST,
        preferred_element_type=jnp.float32,
    )


def _tc_dense_body(agg_ref, deg_ref, xs_ref, wz_ref, bz_ref, wh_ref, bh_ref,
                   lwz_ref, lbz_ref, lwh_ref, lbh_ref, out_ref):
    dinv = lax.rsqrt(_deg_of(deg_ref))
    p = dinv * (agg_ref[0] + agg_ref[1] + xs_ref[...])
    az = lwz_ref[:D, :]
    ah = lwh_ref[:D, :]
    mz = _dot(wz_ref[...], az)
    mh = _dot(wh_ref[...], ah)
    cz = _dot(bz_ref[...], az) + lbz_ref[...]
    ch = _dot(bh_ref[...], ah) + lbh_ref[...]
    z = jax.nn.sigmoid(_dot(p, mz) + cz)
    ht = jnp.tanh(_dot(p, mh) + ch)
    out_ref[...] = (1.0 - z) * ht


def _tc_dense(agg_parts, deg_parts, Xs, W_z, b_z, W_h, b_h, LW_z, Lb_z, LW_h, Lb_h):
    def full(shape):
        return pl.BlockSpec(shape, lambda i: tuple(0 for _ in shape))

    return pl.pallas_call(
        _tc_dense_body,
        grid=(_GRID,),
        in_specs=[
            pl.BlockSpec((NC, _BLK, D), lambda i: (0, i, 0)),
            pl.BlockSpec((NC, _BLK, D), lambda i: (0, i, 0)),
            pl.BlockSpec((_BLK, D), lambda i: (i, 0)),
            full((D, D)),
            full((1, D)),
            full((D, D)),
            full((1, D)),
            full((2 * D, D)),
            full((1, D)),
            full((2 * D, D)),
            full((1, D)),
        ],
        out_specs=pl.BlockSpec((_BLK, D), lambda i: (i, 0)),
        out_shape=jax.ShapeDtypeStruct((N, D), jnp.float32),
    )(agg_parts, deg_parts, Xs, W_z, b_z, W_h, b_h, LW_z, Lb_z, LW_h, Lb_h)


def kernel(X, edge_index, edge_weight, W_z, b_z, W_r, b_r, W_h, b_h,
           LW_z, Lb_z, LW_r, Lb_r, LW_h, Lb_h):
    pad = E_PAD - E
    row_p = jnp.concatenate([edge_index[0], jnp.zeros((pad,), edge_index.dtype)])
    col_p = jnp.concatenate([edge_index[1], jnp.zeros((pad,), edge_index.dtype)])
    ew_p = jnp.concatenate([edge_weight, jnp.zeros((pad,), edge_weight.dtype)])
    row2d = row_p.reshape(NW * NCH, CHUNK)
    col2d = col_p.reshape(NW * NCH, CHUNK)
    ew2d = ew_p.reshape(NW * NCH, CHUNK)
    deg_parts = _sc_deg(col2d, ew2d)
    Xs = _tc_prescale(deg_parts, X)
    agg_parts = _sc_agg(row2d, col2d, ew2d, Xs)
    return _tc_dense(
        agg_parts, deg_parts, Xs,
        W_z, b_z.reshape(1, D), W_h, b_h.reshape(1, D),
        LW_z, Lb_z.reshape(1, D), LW_h, Lb_h.reshape(1, D),
    )


# speedup vs baseline: 26.9265x; 1.0463x over previous
import jax
import jax.numpy as jnp
from jax import lax
from jax.experimental import pallas as pl
from jax.experimental.pallas import tpu as pltpu
from jax.experimental.pallas import tpu_sc as plsc

N = 10000
E = 320000
D = 128
N_PAD = 10240
NC = 2
NS = 16
NW = NC * NS
CHUNK = 80
NCH = 126
E_PAD = NW * NCH * CHUNK
ROWS_PER_TILE = N_PAD // NS
NPAIR = NCH // 2
NCH_C0 = 174
NCH_C1 = 78

_mesh = plsc.VectorSubcoreMesh(core_axis_name="c", subcore_axis_name="s")

_BCAST_DN = lax.GatherDimensionNumbers(
    offset_dims=(), collapsed_slice_dims=(0,), start_index_map=(0,))


def _bcast(vec, i):
    return lax.gather(
        vec, jnp.full((16, 1), i, jnp.int32), _BCAST_DN, (1,),
        mode=lax.GatherScatterMode.PROMISE_IN_BOUNDS)


def _zero_vmem(buf, rows):
    for r in range(rows):
        for db in range(D // 16):
            buf[r, pl.ds(db * 16, 16)] = jnp.zeros((16,), jnp.float32)


def _zero_shared(zbuf, shared, sub):
    _zero_vmem(zbuf, 16)
    for t in range(ROWS_PER_TILE // 16):
        pltpu.sync_copy(zbuf, shared.at[pl.ds(sub * ROWS_PER_TILE + t * 16, 16)])


def _dump_shared(zbuf, shared, out_hbm, core, sub):
    for t in range(ROWS_PER_TILE // 16):
        base = sub * ROWS_PER_TILE + t * 16
        pltpu.sync_copy(shared.at[pl.ds(base, 16)], zbuf)
        pltpu.sync_copy(zbuf, out_hbm.at[core, pl.ds(base, 16)])


def _regcopy80(src, dst):
    for k in range(CHUNK // 16):
        sl = pl.ds(k * 16, 16)
        dst[0, sl] = src[0, sl]


def _sc_deg(col2d, ew2d):
    def body(col_hbm, ew_hbm, out_hbm, acc_sh,
             col_a, ew_a, cs_a, pay_a, col_b, ew_b, cs_b, pay_b,
             zbuf, isem_a, isem_b, ssem_a, ssem_b):
        c = lax.axis_index("c")
        s = lax.axis_index("s")
        wid = c * NS + s
        _zero_vmem(pay_a, CHUNK)
        _zero_vmem(pay_b, CHUNK)
        _zero_shared(zbuf, acc_sh, s)
        plsc.subcore_barrier()

        io = lax.iota(jnp.int32, 16)
        zz = jnp.zeros((16,), jnp.float32)

        def idx_issue(j, cv, ev, sem):
            base = wid * NCH + j
            pltpu.async_copy(col_hbm.at[pl.ds(base, 1)], cv, sem)
            pltpu.async_copy(ew_hbm.at[pl.ds(base, 1)], ev, sem)

        def idx_wait(cv, ev, sem):
            pltpu.make_async_copy(col_hbm.at[pl.ds(0, 1)], cv, sem).wait()
            pltpu.make_async_copy(ew_hbm.at[pl.ds(0, 1)], ev, sem).wait()

        def build(ev, pv):
            for g in range(CHUNK // 16):
                ewg = ev[0, pl.ds(g * 16, 16)]
                for i in range(16):
                    pv[g * 16 + i, pl.ds(0, 16)] = jnp.where(io == i, ewg, zz)

        def scat_issue(pv, csv, sem):
            pltpu.async_copy(pv, acc_sh.at[csv.at[0, :]], sem, add=True)

        def scat_wait(pv, csv, sem):
            pltpu.make_async_copy(pv, acc_sh.at[csv.at[0, :]], sem).wait()

        idx_issue(0, col_a, ew_a, isem_a)
        idx_issue(1, col_b, ew_b, isem_b)
        idx_wait(col_a, ew_a, isem_a)
        idx_wait(col_b, ew_b, isem_b)

        def pair(jj, carry):
            j0 = 2 * jj
            build(ew_a, pay_a)
            _regcopy80(col_a, cs_a)
            scat_issue(pay_a, cs_a, ssem_a)
            build(ew_b, pay_b)
            _regcopy80(col_b, cs_b)
            scat_wait(pay_a, cs_a, ssem_a)
            pltpu.sync_copy(pay_b, acc_sh.at[cs_b.at[0, :]], add=True)

            @pl.when(jj < NPAIR - 1)
            def _():
                idx_issue(j0 + 2, col_a, ew_a, isem_a)
                idx_issue(j0 + 3, col_b, ew_b, isem_b)
                idx_wait(col_a, ew_a, isem_a)
                idx_wait(col_b, ew_b, isem_b)

            return carry

        lax.fori_loop(0, NPAIR, pair, 0)
        plsc.subcore_barrier()
        _dump_shared(zbuf, acc_sh, out_hbm, c, s)

    k = pl.kernel(
        body,
        out_type=jax.ShapeDtypeStruct((NC, N_PAD, D), jnp.float32),
        mesh=_mesh,
        scratch_types=[
            pltpu.VMEM_SHARED((N_PAD, D), jnp.float32),
            pltpu.VMEM((1, CHUNK), jnp.int32),
            pltpu.VMEM((1, CHUNK), jnp.float32),
            pltpu.VMEM((1, CHUNK), jnp.int32),
            pltpu.VMEM((CHUNK, D), jnp.float32),
            pltpu.VMEM((1, CHUNK), jnp.int32),
            pltpu.VMEM((1, CHUNK), jnp.float32),
            pltpu.VMEM((1, CHUNK), jnp.int32),
            pltpu.VMEM((CHUNK, D), jnp.float32),
            pltpu.VMEM((16, D), jnp.float32),
            pltpu.SemaphoreType.DMA,
            pltpu.SemaphoreType.DMA,
            pltpu.SemaphoreType.DMA,
            pltpu.SemaphoreType.DMA,
        ],
    )
    return k(col2d, ew2d)


def _sc_agg(row2d, col2d, ew2d, Xs):
    def body(row_hbm, col_hbm, ew_hbm, xs_hbm, out_hbm, acc_sh,
             row_a, col_a, ew_a, cs_a, rows_a,
             row_b, col_b, ew_b, cs_b, rows_b,
             zbuf, isem_a, isem_b, gsem_a, gsem_b, ssem_a, ssem_b):
        c = lax.axis_index("c")
        s = lax.axis_index("s")
        _zero_shared(zbuf, acc_sh, s)
        plsc.subcore_barrier()
        slab = jnp.where(c == 0, s * NCH_C0, NS * NCH_C0 + s * NCH_C1)
        npair = jnp.where(c == 0, NCH_C0 // 2, NCH_C1 // 2)

        def idx_issue(j, rv, cv, ev, sem):
            base = slab + j
            pltpu.async_copy(row_hbm.at[pl.ds(base, 1)], rv, sem)
            pltpu.async_copy(col_hbm.at[pl.ds(base, 1)], cv, sem)
            pltpu.async_copy(ew_hbm.at[pl.ds(base, 1)], ev, sem)

        def idx_wait(rv, cv, ev, sem):
            pltpu.make_async_copy(row_hbm.at[pl.ds(0, 1)], rv, sem).wait()
            pltpu.make_async_copy(col_hbm.at[pl.ds(0, 1)], cv, sem).wait()
            pltpu.make_async_copy(ew_hbm.at[pl.ds(0, 1)], ev, sem).wait()

        def gat_issue(rv, dst, sem):
            pltpu.async_copy(xs_hbm.at[rv.at[0, :]], dst, sem)

        def gat_wait(rv, dst, sem):
            pltpu.make_async_copy(xs_hbm.at[rv.at[0, :]], dst, sem).wait()

        def scale(ev, rowsv):
            for g in range(CHUNK // 16):
                ewg = ev[0, pl.ds(g * 16, 16)]
                for i in range(16):
                    e = g * 16 + i
                    sv = _bcast(ewg, i)
                    for db in range(D // 16):
                        sl = pl.ds(db * 16, 16)
                        rowsv[e, sl] = rowsv[e, sl] * sv

        def scat_issue(rowsv, csv, sem):
            pltpu.async_copy(rowsv, acc_sh.at[csv.at[0, :]], sem, add=True)

        def scat_wait(rowsv, csv, sem):
            pltpu.make_async_copy(rowsv, acc_sh.at[csv.at[0, :]], sem).wait()

        idx_issue(0, row_a, col_a, ew_a, isem_a)
        idx_issue(1, row_b, col_b, ew_b, isem_b)
        idx_wait(row_a, col_a, ew_a, isem_a)
        gat_issue(row_a, rows_a, gsem_a)
        idx_wait(row_b, col_b, ew_b, isem_b)
        gat_issue(row_b, rows_b, gsem_b)

        def pair(jj, carry):
            j0 = 2 * jj
            last = jj >= npair - 1
            gat_wait(row_a, rows_a, gsem_a)
            scale(ew_a, rows_a)
            _regcopy80(col_a, cs_a)
            scat_issue(rows_a, cs_a, ssem_a)

            @pl.when(jnp.logical_not(last))
            def _():
                idx_issue(j0 + 2, row_a, col_a, ew_a, isem_a)

            gat_wait(row_b, rows_b, gsem_b)
            scale(ew_b, rows_b)
            _regcopy80(col_b, cs_b)
            scat_wait(rows_a, cs_a, ssem_a)
            pltpu.sync_copy(rows_b, acc_sh.at[cs_b.at[0, :]], add=True)

            @pl.when(jnp.logical_not(last))
            def _():
                idx_issue(j0 + 3, row_b, col_b, ew_b, isem_b)
                idx_wait(row_a, col_a, ew_a, isem_a)
                gat_issue(row_a, rows_a, gsem_a)
                idx_wait(row_b, col_b, ew_b, isem_b)
                gat_issue(row_b, rows_b, gsem_b)

            return carry

        lax.fori_loop(0, npair, pair, 0)
        plsc.subcore_barrier()
        _dump_shared(zbuf, acc_sh, out_hbm, c, s)

    k = pl.kernel(
        body,
        out_type=jax.ShapeDtypeStruct((NC, N_PAD, D), jnp.float32),
        mesh=_mesh,
        scratch_types=[
            pltpu.VMEM_SHARED((N_PAD, D), jnp.float32),
            pltpu.VMEM((1, CHUNK), jnp.int32),
            pltpu.VMEM((1, CHUNK), jnp.int32),
            pltpu.VMEM((1, CHUNK), jnp.float32),
            pltpu.VMEM((1, CHUNK), jnp.int32),
            pltpu.VMEM((CHUNK, D), jnp.float32),
            pltpu.VMEM((1, CHUNK), jnp.int32),
            pltpu.VMEM((1, CHUNK), jnp.int32),
            pltpu.VMEM((1, CHUNK), jnp.float32),
            pltpu.VMEM((1, CHUNK), jnp.int32),
            pltpu.VMEM((CHUNK, D), jnp.float32),
            pltpu.VMEM((16, D), jnp.float32),
            pltpu.SemaphoreType.DMA,
            pltpu.SemaphoreType.DMA,
            pltpu.SemaphoreType.DMA,
            pltpu.SemaphoreType.DMA,
            pltpu.SemaphoreType.DMA,
            pltpu.SemaphoreType.DMA,
        ],
    )
    return k(row2d, col2d, ew2d, Xs)


_BLK = 1000
_GRID = N // _BLK


def _deg_of(deg_ref):
    return (1.0 + jnp.sum(deg_ref[0, :, :16], axis=-1, keepdims=True)
            + jnp.sum(deg_ref[1, :, :16], axis=-1, keepdims=True))


def _tc_prescale_body(deg_ref, x_ref, xs_ref):
    dinv = lax.rsqrt(_deg_of(deg_ref))
    xs_ref[...] = x_ref[...] * dinv


def _tc_prescale(deg_parts, X):
    return pl.pallas_call(
        _tc_prescale_body,
        grid=(_GRID,),
        in_specs=[
            pl.BlockSpec((NC, _BLK, D), lambda i: (0, i, 0)),
            pl.BlockSpec((_BLK, D), lambda i: (i, 0)),
        ],
        out_specs=pl.BlockSpec((_BLK, D), lambda i: (i, 0)),
        out_shape=jax.ShapeDtypeStruct((N, D), jnp.float32),
    )(deg_parts, X)


def _dot(a, b):
    return lax.dot_general(
        a, b, (((1,), (0,)), ((), ())),
        precision=lax.Precision.HIGHEST,
        preferred_element_type=jnp.float32,
    )


def _tc_dense_body(agg_ref, deg_ref, xs_ref, wz_ref, bz_ref, wh_ref, bh_ref,
                   lwz_ref, lbz_ref, lwh_ref, lbh_ref, out_ref):
    dinv = lax.rsqrt(_deg_of(deg_ref))
    p = dinv * (agg_ref[0] + agg_ref[1] + xs_ref[...])
    az = lwz_ref[:D, :]
    ah = lwh_ref[:D, :]
    mz = _dot(wz_ref[...], az)
    mh = _dot(wh_ref[...], ah)
    cz = _dot(bz_ref[...], az) + lbz_ref[...]
    ch = _dot(bh_ref[...], ah) + lbh_ref[...]
    z = jax.nn.sigmoid(_dot(p, mz) + cz)
    ht = jnp.tanh(_dot(p, mh) + ch)
    out_ref[...] = (1.0 - z) * ht


def _tc_dense(agg_parts, deg_parts, Xs, W_z, b_z, W_h, b_h, LW_z, Lb_z, LW_h, Lb_h):
    def full(shape):
        return pl.BlockSpec(shape, lambda i: tuple(0 for _ in shape))

    return pl.pallas_call(
        _tc_dense_body,
        grid=(_GRID,),
        in_specs=[
            pl.BlockSpec((NC, _BLK, D), lambda i: (0, i, 0)),
            pl.BlockSpec((NC, _BLK, D), lambda i: (0, i, 0)),
            pl.BlockSpec((_BLK, D), lambda i: (i, 0)),
            full((D, D)),
            full((1, D)),
            full((D, D)),
            full((1, D)),
            full((2 * D, D)),
            full((1, D)),
            full((2 * D, D)),
            full((1, D)),
        ],
        out_specs=pl.BlockSpec((_BLK, D), lambda i: (i, 0)),
        out_shape=jax.ShapeDtypeStruct((N, D), jnp.float32),
    )(agg_parts, deg_parts, Xs, W_z, b_z, W_h, b_h, LW_z, Lb_z, LW_h, Lb_h)


def kernel(X, edge_index, edge_weight, W_z, b_z, W_r, b_r, W_h, b_h,
           LW_z, Lb_z, LW_r, Lb_r, LW_h, Lb_h):
    pad = E_PAD - E
    row_p = jnp.concatenate([edge_index[0], jnp.zeros((pad,), edge_index.dtype)])
    col_p = jnp.concatenate([edge_index[1], jnp.zeros((pad,), edge_index.dtype)])
    ew_p = jnp.concatenate([edge_weight, jnp.zeros((pad,), edge_weight.dtype)])
    row2d = row_p.reshape(NW * NCH, CHUNK)
    col2d = col_p.reshape(NW * NCH, CHUNK)
    ew2d = ew_p.reshape(NW * NCH, CHUNK)
    deg_parts = _sc_deg(col2d, ew2d)
    Xs = _tc_prescale(deg_parts, X)
    agg_parts = _sc_agg(row2d, col2d, ew2d, Xs)
    return _tc_dense(
        agg_parts, deg_parts, Xs,
        W_z, b_z.reshape(1, D), W_h, b_h.reshape(1, D),
        LW_z, Lb_z.reshape(1, D), LW_h, Lb_h.reshape(1, D),
    )
